# speedup vs baseline: 57.1099x; 57.1099x over previous
import functools

import jax
import jax.numpy as jnp
from jax.experimental import pallas as pl
from jax.experimental.pallas import tpu as pltpu

_N = 32
_KT = 8192


def _nms_tc_kernel(thr_ref, x_ref, out_ref, acc_ref, *, nsteps):
    k = pl.program_id(0)

    @pl.when(k == 0)
    def _():
        acc_ref[...] = jnp.zeros_like(acc_ref)

    x = x_ref[...]
    acc_ref[...] += jax.lax.dot_general(
        x, x, (((1,), (1,)), ((), ())), preferred_element_type=jnp.float32
    )

    @pl.when(k == nsteps - 1)
    def _():
        g = acc_ref[...]
        thr = thr_ref[0]
        row_id = jax.lax.broadcasted_iota(jnp.int32, (_N, _N), 0)
        col_id = jax.lax.broadcasted_iota(jnp.int32, (_N, _N), 1)
        gdiag = jnp.where(row_id == col_id, g, 0.0)
        score_col = jnp.sum(gdiag, axis=1, keepdims=True)
        score_row = jnp.sum(gdiag, axis=0, keepdims=True)
        iou = g / (score_col + score_row - g)
        acc_ref[...] = iou
        lane = jax.lax.broadcasted_iota(jnp.int32, (1, _N), 1)

        def body(i, ind):
            row = acc_ref[pl.ds(i, 1), :]
            is_i = lane == i
            active = jnp.sum(jnp.where(is_i, ind, 0.0)) > 0.0
            score_i = jnp.sum(jnp.where(is_i, score_row, 0.0))
            hit = jnp.logical_and(active, row > thr)
            killv = jnp.logical_and(hit, score_i > score_row)
            jstar = jnp.min(jnp.where(killv, lane, _N))
            sup = jnp.logical_and(
                hit, jnp.logical_and(score_row > score_i, lane <= jstar)
            )
            suppress_i = jnp.max(jnp.where(sup, 1, 0)) > 0
            ind = jnp.where(jnp.logical_and(is_i, suppress_i), 0.0, ind)
            ind = jnp.where(lane == jstar, 0.0, ind)
            return ind

        ind = jax.lax.fori_loop(0, _N, body, jnp.ones((1, _N), jnp.float32))
        out_ref[...] = ind


@jax.jit
def kernel(mask, threshold):
    flat = mask.reshape(_N, -1)
    v = flat.shape[1]
    nsteps = v // _KT
    thr = jnp.asarray(threshold, jnp.float32).reshape(1)
    out = pl.pallas_call(
        functools.partial(_nms_tc_kernel, nsteps=nsteps),
        grid=(nsteps,),
        in_specs=[
            pl.BlockSpec(memory_space=pltpu.SMEM),
            pl.BlockSpec((_N, _KT), lambda k: (0, k)),
        ],
        out_specs=pl.BlockSpec((1, _N), lambda k: (0, 0)),
        out_shape=jax.ShapeDtypeStruct((1, _N), jnp.float32),
        scratch_shapes=[pltpu.VMEM((_N, _N), jnp.float32)],
    )(thr, flat)
    return out.reshape(_N) > 0


# speedup vs baseline: 135.7778x; 2.3775x over previous
import functools

import jax
import jax.numpy as jnp
from jax import lax
from jax.experimental import pallas as pl
from jax.experimental.pallas import tpu as pltpu
from jax.experimental.pallas import tpu_sc as plsc

_N = 32
_LANES = 128
_BC = 256


def _gram_kernel(thr_ref, x_ref, kill_ref, sup_ref, acc_ref, *, nsteps):
    k = pl.program_id(0)

    @pl.when(k == 0)
    def _():
        acc_ref[...] = jnp.zeros_like(acc_ref)

    rowg = lax.broadcasted_iota(jnp.int32, (8 * _N, 8 * _N), 0) % 8
    colg = lax.broadcasted_iota(jnp.int32, (8 * _N, 8 * _N), 1) % 8
    diag8 = rowg == colg
    e_row = lax.broadcasted_iota(jnp.int32, (8 * _N, _N), 0) // 8
    e_col = lax.broadcasted_iota(jnp.int32, (8 * _N, _N), 1)
    e = jnp.where(e_row == e_col, 1.0, 0.0).astype(jnp.bfloat16)
    s_acc = jnp.zeros((_N, 8 * _N), jnp.float32)
    for g in range(_BC // 8):
        z = x_ref[:, g * 8:(g + 1) * 8, :].reshape(8 * _N, _LANES)
        zb = z.astype(jnp.bfloat16)
        h = lax.dot_general(
            zb, zb, (((1,), (1,)), ((), ())), preferred_element_type=jnp.float32
        )
        hm = jnp.where(diag8, h, 0.0).astype(jnp.bfloat16)
        s_acc += lax.dot_general(
            e, hm, (((0,), (0,)), ((), ())), preferred_element_type=jnp.float32
        )
    acc_ref[...] += s_acc

    @pl.when(k == nsteps - 1)
    def _():
        s = acc_ref[...]
        sel_row = lax.broadcasted_iota(jnp.int32, (8 * _N, _N), 0) // 8
        sel_col = lax.broadcasted_iota(jnp.int32, (8 * _N, _N), 1)
        p = jnp.where(sel_row == sel_col, 1.0, 0.0)
        g = lax.dot_general(
            s, p, (((1,), (0,)), ((), ())),
            preferred_element_type=jnp.float32,
            precision=lax.Precision.HIGHEST,
        )
        thr = thr_ref[0]
        row_id = lax.broadcasted_iota(jnp.int32, (_N, _N), 0)
        col_id = lax.broadcasted_iota(jnp.int32, (_N, _N), 1)
        gdiag = jnp.where(row_id == col_id, g, 0.0)
        score_col = jnp.sum(gdiag, axis=1, keepdims=True)
        score_row = jnp.sum(gdiag, axis=0, keepdims=True)
        iou = g / (score_col + score_row - g)
        hit = iou > thr
        killf = jnp.where(hit & (score_col > score_row), 1.0, 0.0)
        supf = jnp.where(hit & (score_row > score_col), 1.0, 0.0)
        r8 = lax.broadcasted_iota(jnp.int32, (8, _N), 0)
        c8 = lax.broadcasted_iota(jnp.int32, (8, _N), 1)

        def pack(m):
            parts = []
            for q in range(4):
                sq = jnp.where(c8 == 4 * r8 + q, 1.0, 0.0)
                parts.append(
                    lax.dot_general(
                        sq, m, (((1,), (0,)), ((), ())),
                        preferred_element_type=jnp.float32,
                    )
                )
            return jnp.concatenate(parts, axis=1).astype(jnp.int32)

        kill_ref[...] = pack(killf)
        sup_ref[...] = pack(supf)


def _sc_nms_body(kill_hbm, sup_hbm, out_hbm, kill_v, sup_v, out_v):
    cid = lax.axis_index("c")
    sid = lax.axis_index("s")

    @pl.when(jnp.logical_and(cid == 0, sid == 0))
    def _():
        pltpu.sync_copy(kill_hbm, kill_v)
        pltpu.sync_copy(sup_hbm, sup_v)
        lane = lax.iota(jnp.int32, 16)
        lane_hi = lane + 16
        ind_lo = jnp.full((16,), 1, jnp.int32)
        ind_hi = jnp.full((16,), 1, jnp.int32)
        for i in range(_N):
            is_i = lane == (i % 16)
            ind_half = ind_lo if i < 16 else ind_hi
            active = plsc.all_reduce_population_count(is_i & (ind_half != 0)) > 0
            r, c0 = i // 4, (i % 4) * 32
            kr_lo = kill_v[r, pl.ds(c0, 16)]
            kr_hi = kill_v[r, pl.ds(c0 + 16, 16)]
            sr_lo = sup_v[r, pl.ds(c0, 16)]
            sr_hi = sup_v[r, pl.ds(c0 + 16, 16)]
            ffs_lo = plsc.all_reduce_ffs(kr_lo != 0)
            ffs_hi = plsc.all_reduce_ffs(kr_hi != 0)
            jstar = jnp.where(ffs_lo < 16, ffs_lo, 16 + ffs_hi)
            jstar = jnp.where(active, jstar, _N)
            sup_n = plsc.all_reduce_population_count(
                (sr_lo != 0) & (lane <= jstar)
            ) + plsc.all_reduce_population_count((sr_hi != 0) & (lane_hi <= jstar))
            suppress = active & (sup_n > 0)
            if i < 16:
                ind_lo = jnp.where(is_i & suppress, 0, ind_lo)
            else:
                ind_hi = jnp.where(is_i & suppress, 0, ind_hi)
            ind_lo = jnp.where(lane == jstar, 0, ind_lo)
            ind_hi = jnp.where(lane_hi == jstar, 0, ind_hi)
        out_v[pl.ds(0, 16)] = ind_lo
        out_v[pl.ds(16, 16)] = ind_hi
        pltpu.sync_copy(out_v, out_hbm)


_sc_nms = pl.kernel(
    _sc_nms_body,
    out_type=jax.ShapeDtypeStruct((_N,), jnp.int32),
    mesh=plsc.VectorSubcoreMesh(core_axis_name="c", subcore_axis_name="s"),
    compiler_params=pltpu.CompilerParams(needs_layout_passes=False),
    scratch_types=[
        pltpu.VMEM((8, 128), jnp.int32),
        pltpu.VMEM((8, 128), jnp.int32),
        pltpu.VMEM((_N,), jnp.int32),
    ],
)


def _tc_part(mask, threshold):
    x = jnp.transpose(mask, (0, 1, 2, 4, 3)).reshape(_N, -1, _LANES)
    nsteps = x.shape[1] // _BC
    thr = jnp.asarray(threshold, jnp.float32).reshape(1)
    kill, sup = pl.pallas_call(
        functools.partial(_gram_kernel, nsteps=nsteps),
        grid=(nsteps,),
        in_specs=[
            pl.BlockSpec(memory_space=pltpu.SMEM),
            pl.BlockSpec((_N, _BC, _LANES), lambda k: (0, k, 0)),
        ],
        out_specs=[
            pl.BlockSpec((8, 128), lambda k: (0, 0)),
            pl.BlockSpec((8, 128), lambda k: (0, 0)),
        ],
        out_shape=[
            jax.ShapeDtypeStruct((8, 128), jnp.int32),
            jax.ShapeDtypeStruct((8, 128), jnp.int32),
        ],
        scratch_shapes=[pltpu.VMEM((_N, 8 * _N), jnp.float32)],
    )(thr, x)
    return kill, sup


@jax.jit
def kernel(mask, threshold):
    kill, sup = _tc_part(mask, threshold)
    return _sc_nms(kill, sup) > 0
